# baseline (speedup 1.0000x reference)
import functools

import jax
import jax.numpy as jnp
from jax import lax
from jax.experimental import pallas as pl
from jax.experimental.pallas import tpu as pltpu
from jax.experimental.pallas import tpu_sc as plsc

_NC = 2
_NS = 16
_LANES = 16

_CHUNK = 1024
_SUB = 128
_NSUB = _CHUNK // _SUB
_ZROWS = 640
_NRANGE = 2

_G = 64
_BN = 1000


def _ceil_to(a, m):
    return ((a + m - 1) // m) * m


def _fill_const(ref, rows, value):
    def body(r, carry):
        ref[r] = jnp.full((_LANES,), value, jnp.float32)
        return carry

    lax.fori_loop(0, rows, body, None)


def _zero_acc(acc, zerov, span, tile):
    def body(z, carry):
        pltpu.sync_copy(zerov, acc.at[pl.ds(tile * span + z * _ZROWS, _ZROWS)])
        return carry

    lax.fori_loop(0, span // _ZROWS, body, None)


def _clamped_scatter_idx(srcv, sidx, n2, base):
    for j in range(_NSUB):
        def body(u, carry):
            sl = srcv[pl.ds(j * _SUB + u * _LANES, _LANES)] - base
            ok = (sl >= 0) & (sl < n2)
            sidx[j, pl.ds(u * _LANES, _LANES)] = jnp.where(ok, sl, n2)
            return carry

        lax.fori_loop(0, _SUB // _LANES, body, None)


def _scatter_chunk(rowsv, sidx, acc, sem):
    waits = []
    for j in range(_NSUB):
        waits.append(pltpu.async_copy(
            rowsv.at[pl.ds(j * _SUB, _SUB)], acc.at[sidx.at[j]], sem,
            add=True))
    return waits


@functools.lru_cache(maxsize=None)
def _build_sc_degree(ep, n2, acc_rows):
    per_tile = ep // (_NC * _NS)
    n_chunks = per_tile // _CHUNK
    span = acc_rows // _NS
    mesh = plsc.VectorSubcoreMesh(core_axis_name="c", subcore_axis_name="s")

    @functools.partial(
        pl.kernel,
        out_type=jax.ShapeDtypeStruct((_NC, _NRANGE, acc_rows, _LANES),
                                      jnp.float32),
        mesh=mesh,
        scratch_types=[
            pltpu.VMEM((_CHUNK,), jnp.int32),
            pltpu.VMEM((_CHUNK,), jnp.int32),
            pltpu.VMEM((_NSUB, _SUB), jnp.int32),
            pltpu.VMEM((_NSUB, _SUB), jnp.int32),
            pltpu.VMEM((_SUB, _LANES), jnp.float32),
            pltpu.VMEM((_ZROWS, _LANES), jnp.float32),
            pltpu.VMEM_SHARED((acc_rows, _LANES), jnp.float32),
            pltpu.SemaphoreType.DMA,
        ],
        compiler_params=pltpu.CompilerParams(use_tc_tiling_on_sc=False),
    )
    def deg_kernel(src_hbm, out_hbm, srcv0, srcv1, sidx0, sidx1,
                   onesv, zerov, acc, sem_s):
        c = lax.axis_index("c")
        s = lax.axis_index("s")
        _fill_const(onesv, _SUB, 1.0)
        _fill_const(zerov, _ZROWS, 0.0)
        ebase = c * (ep // _NC) + s * per_tile

        for r in range(_NRANGE):
            _zero_acc(acc, zerov, span, s)
            plsc.subcore_barrier()

            def pair(ci, carry):
                base = ebase + ci * (2 * _CHUNK)
                waits = []
                for b, (srcv, sidx) in enumerate(((srcv0, sidx0),
                                                  (srcv1, sidx1))):
                    pltpu.sync_copy(
                        src_hbm.at[pl.ds(base + b * _CHUNK, _CHUNK)], srcv)
                    _clamped_scatter_idx(srcv, sidx, n2, r * n2)
                    for j in range(_NSUB):
                        waits.append(pltpu.async_copy(
                            onesv, acc.at[sidx.at[j]], sem_s, add=True))
                for w in waits:
                    w.wait()
                return carry

            lax.fori_loop(0, n_chunks // 2, pair, None)
            plsc.subcore_barrier()
            pltpu.sync_copy(
                acc.at[pl.ds(s * span, span)],
                out_hbm.at[c, r, pl.ds(s * span, span)],
            )
            plsc.subcore_barrier()

    return deg_kernel


@functools.lru_cache(maxsize=None)
def _build_sc_segsum(ep, n2, k_groups, acc_rows):
    per_tile = ep // _NS
    n_chunks = per_tile // _CHUNK
    span = acc_rows // _NS
    gpc = k_groups // _NC
    mesh = plsc.VectorSubcoreMesh(core_axis_name="c", subcore_axis_name="s")

    @functools.partial(
        pl.kernel,
        out_type=jax.ShapeDtypeStruct((k_groups, _NRANGE, acc_rows, _LANES),
                                      jnp.float32),
        mesh=mesh,
        scratch_types=[
            pltpu.VMEM((_CHUNK,), jnp.int32),
            pltpu.VMEM((_CHUNK,), jnp.int32),
            pltpu.VMEM((_CHUNK,), jnp.int32),
            pltpu.VMEM((_CHUNK,), jnp.int32),
            pltpu.VMEM((_CHUNK,), jnp.int32),
            pltpu.VMEM((_CHUNK,), jnp.int32),
            pltpu.VMEM((_NSUB, _SUB), jnp.int32),
            pltpu.VMEM((_NSUB, _SUB), jnp.int32),
            pltpu.VMEM((_CHUNK, _LANES), jnp.float32),
            pltpu.VMEM((_CHUNK, _LANES), jnp.float32),
            pltpu.VMEM((_ZROWS, _LANES), jnp.float32),
            pltpu.VMEM_SHARED((acc_rows, _LANES), jnp.float32),
            pltpu.SemaphoreType.DMA,
            pltpu.SemaphoreType.DMA,
            pltpu.SemaphoreType.DMA,
        ],
        compiler_params=pltpu.CompilerParams(use_tc_tiling_on_sc=False),
    )
    def seg_kernel(p_hbm, dst_hbm, src_hbm, out_hbm,
                   dstv0, dstv1, srcv0, srcv1, gidx0, gidx1, sidx0, sidx1,
                   rows0, rows1, zerov, acc, sem_g0, sem_g1, sem_s):
        c = lax.axis_index("c")
        s = lax.axis_index("s")
        _fill_const(zerov, _ZROWS, 0.0)
        ebase = s * per_tile
        slots = ((dstv0, srcv0, gidx0, sidx0, rows0, sem_g0),
                 (dstv1, srcv1, gidx1, sidx1, rows1, sem_g1))

        for gi in range(gpc):
            g = c * gpc + gi
            for r in range(_NRANGE):
                _zero_acc(acc, zerov, span, s)
                plsc.subcore_barrier()

                def pair(ci, carry):
                    base = ebase + ci * (2 * _CHUNK)
                    gathers = []
                    for b, (dstv, srcv, gidxv, sidx, rowsv, sg) in \
                            enumerate(slots):
                        cb = base + b * _CHUNK
                        pltpu.sync_copy(dst_hbm.at[pl.ds(cb, _CHUNK)], dstv)
                        pltpu.sync_copy(src_hbm.at[pl.ds(cb, _CHUNK)], srcv)

                        def mk_idx(u, carry2, dstv=dstv, gidxv=gidxv):
                            o = u * _LANES
                            gidxv[pl.ds(o, _LANES)] = (
                                dstv[pl.ds(o, _LANES)] * k_groups + g)
                            return carry2

                        lax.fori_loop(0, _CHUNK // _LANES, mk_idx, None)
                        _clamped_scatter_idx(srcv, sidx, n2, r * n2)
                        gathers.append(pltpu.async_copy(
                            p_hbm.at[gidxv], rowsv, sg))
                    scatters = []
                    for b, (dstv, srcv, gidxv, sidx, rowsv, sg) in \
                            enumerate(slots):
                        gathers[b].wait()
                        scatters += _scatter_chunk(rowsv, sidx, acc, sem_s)
                    for w in scatters:
                        w.wait()
                    return carry

                lax.fori_loop(0, n_chunks // 2, pair, None)
                plsc.subcore_barrier()
                pltpu.sync_copy(
                    acc.at[pl.ds(s * span, span)],
                    out_hbm.at[g, r, pl.ds(s * span, span)],
                )
                plsc.subcore_barrier()

    return seg_kernel


def _k1_body(degp_ref, x_ref, dinv_ref, p1_ref):
    deg = degp_ref[0, 0, :, 0:1] + degp_ref[1, 0, :, 0:1]
    dinv = lax.rsqrt(jnp.maximum(deg, 1.0))
    dinv_ref[...] = dinv
    bn, f = x_ref.shape
    fp = p1_ref.shape[1]
    p1_ref[...] = jnp.concatenate(
        [x_ref[...] * dinv, jnp.zeros((bn, fp - f), jnp.float32)], axis=1)


def _k2_body(agg1_ref, dinv_ref, x_ref, w1a_ref, w1b_ref, b1_ref, w2a_ref,
             h1_ref, p2_ref):
    dv = dinv_ref[...]
    a = jnp.concatenate([agg1_ref[kk, 0] for kk in range(2)], axis=1) * dv
    h1 = jnp.dot(a, w1a_ref[...], preferred_element_type=jnp.float32)
    h1 += jnp.dot(x_ref[...], w1b_ref[...], preferred_element_type=jnp.float32)
    h1 = jnp.maximum(h1 + b1_ref[...], 0.0)
    h1_ref[...] = h1
    p2_ref[...] = jnp.dot(h1, w2a_ref[...],
                          preferred_element_type=jnp.float32) * dv


def _k3_body(agg2_ref, dinv_ref, h1_ref, w2b_ref, b2_ref, w3a_ref,
             h2_ref, p3_ref):
    dv = dinv_ref[...]
    h2 = jnp.concatenate([agg2_ref[kk, 0] for kk in range(4)], axis=1) * dv
    h2 += jnp.dot(h1_ref[...], w2b_ref[...], preferred_element_type=jnp.float32)
    h2 = jnp.maximum(h2 + b2_ref[...], 0.0)
    h2_ref[...] = h2
    p3_ref[...] = jnp.dot(h2, w3a_ref[...],
                          preferred_element_type=jnp.float32) * dv


def _k4_body(agg3_ref, dinv_ref, h2_ref, w3b_ref, b3_ref, i_ref,
             s_ref, cnt_ref):
    nb = pl.program_id(0)

    @pl.when(nb == 0)
    def _init():
        s_ref[...] = jnp.zeros_like(s_ref)
        cnt_ref[...] = jnp.zeros_like(cnt_ref)

    h3 = (jnp.concatenate([agg3_ref[kk, 0] for kk in range(2)], axis=1)
          * dinv_ref[...])
    h3 += jnp.dot(h2_ref[...], w3b_ref[...], preferred_element_type=jnp.float32)
    h3 = jnp.maximum(h3 + b3_ref[...], 0.0)
    ib = i_ref[0]
    oh = (lax.broadcasted_iota(jnp.int32, (_G, _BN), 0) == ib).astype(jnp.float32)
    s_ref[...] += jnp.dot(oh, h3, preferred_element_type=jnp.float32)
    cnt_ref[...] += jnp.dot(oh, jnp.ones((_BN, 1), jnp.float32),
                            preferred_element_type=jnp.float32)


def _k5_body(s_ref, cnt_ref, wd_ref, bd_ref, out_ref):
    pooled = s_ref[...] / jnp.maximum(cnt_ref[...], 1.0)
    logits = jnp.dot(pooled, wd_ref[...],
                     preferred_element_type=jnp.float32) + bd_ref[...]
    m = jnp.max(logits, axis=1, keepdims=True)
    e = jnp.exp(logits - m)
    out_ref[...] = e / jnp.sum(e, axis=1, keepdims=True)


def _full(shape):
    return pl.BlockSpec(shape, lambda nb: tuple(0 for _ in shape))


def _rows(bn, cols):
    return pl.BlockSpec((bn, cols), lambda nb: (nb, 0))


def _planes(k, bn, nbh):
    return pl.BlockSpec((k, 1, bn, _LANES),
                        lambda nb: (0, nb // nbh, nb % nbh, 0))


def kernel(x, edge_index, i, W1a, W1b, b1, W2a, W2b, b2, W3a, W3b, b3, Wd, bd):
    n, f = x.shape
    e = edge_index.shape[1]
    f1 = W1a.shape[1]
    f2 = W2a.shape[1]
    f3 = W3a.shape[1]
    fp = _ceil_to(f, 2 * _LANES)

    ep = _ceil_to(e, _NC * _NS * _CHUNK * 2)
    n2 = n // _NRANGE
    acc_rows = _ceil_to(n2 + 1, _NS * _ZROWS)
    nb = n // _BN
    nbh = n2 // _BN
    assert n % (_NRANGE * _BN) == 0

    src = edge_index[0]
    dst = edge_index[1]
    src_p = jnp.concatenate([src, jnp.full((ep - e,), n, jnp.int32)])
    dst_p = jnp.concatenate([dst, jnp.full((ep - e,), n, jnp.int32)])

    w1a_p = jnp.pad(W1a, ((0, fp - f), (0, 0)))

    degp = _build_sc_degree(ep, n2, acc_rows)(src_p)

    dinv, p1 = pl.pallas_call(
        _k1_body,
        grid=(nb,),
        in_specs=[_planes(_NC, _BN, nbh), _rows(_BN, f)],
        out_specs=[_rows(_BN, 1), _rows(_BN, fp)],
        out_shape=[jax.ShapeDtypeStruct((n, 1), jnp.float32),
                   jax.ShapeDtypeStruct((n + 1, fp), jnp.float32)],
    )(degp, x)

    k1g = fp // _LANES
    agg1 = _build_sc_segsum(ep, n2, k1g, acc_rows)(
        p1.reshape((n + 1) * k1g, _LANES), dst_p, src_p)

    h1, p2 = pl.pallas_call(
        _k2_body,
        grid=(nb,),
        in_specs=[_planes(k1g, _BN, nbh), _rows(_BN, 1), _rows(_BN, f),
                  _full((fp, f1)), _full((f, f1)), _full((1, f1)),
                  _full((f1, f2))],
        out_specs=[_rows(_BN, f1), _rows(_BN, f2)],
        out_shape=[jax.ShapeDtypeStruct((n, f1), jnp.float32),
                   jax.ShapeDtypeStruct((n + 1, f2), jnp.float32)],
    )(agg1, dinv, x, w1a_p, W1b, b1.reshape(1, f1), W2a)

    k2g = f2 // _LANES
    agg2 = _build_sc_segsum(ep, n2, k2g, acc_rows)(
        p2.reshape((n + 1) * k2g, _LANES), dst_p, src_p)

    h2, p3 = pl.pallas_call(
        _k3_body,
        grid=(nb,),
        in_specs=[_planes(k2g, _BN, nbh), _rows(_BN, 1), _rows(_BN, f1),
                  _full((f1, f2)), _full((1, f2)), _full((f2, f3))],
        out_specs=[_rows(_BN, f2), _rows(_BN, f3)],
        out_shape=[jax.ShapeDtypeStruct((n, f2), jnp.float32),
                   jax.ShapeDtypeStruct((n + 1, f3), jnp.float32)],
    )(agg2, dinv, h1, W2b, b2.reshape(1, f2), W3a)

    k3g = f3 // _LANES
    agg3 = _build_sc_segsum(ep, n2, k3g, acc_rows)(
        p3.reshape((n + 1) * k3g, _LANES), dst_p, src_p)

    s_sum, cnt = pl.pallas_call(
        _k4_body,
        grid=(nb,),
        in_specs=[_planes(k3g, _BN, nbh), _rows(_BN, 1), _rows(_BN, f2),
                  _full((f2, f3)), _full((1, f3)),
                  pl.BlockSpec((1, 1, _BN), lambda nb: (nb, 0, 0))],
        out_specs=[_full((_G, f3)), _full((_G, 1))],
        out_shape=[jax.ShapeDtypeStruct((_G, f3), jnp.float32),
                   jax.ShapeDtypeStruct((_G, 1), jnp.float32)],
    )(agg3, dinv, h2, W3b, b3.reshape(1, f3), i.reshape(nb, 1, _BN))

    out = pl.pallas_call(
        _k5_body,
        grid=(1,),
        in_specs=[_full((_G, f3)), _full((_G, 1)),
                  _full((f3, Wd.shape[1])), _full((1, Wd.shape[1]))],
        out_specs=_full((_G, Wd.shape[1])),
        out_shape=jax.ShapeDtypeStruct((_G, Wd.shape[1]), jnp.float32),
    )(s_sum, cnt, Wd, bd.reshape(1, Wd.shape[1]))

    return out


# speedup vs baseline: 2.1372x; 2.1372x over previous
import functools

import jax
import jax.numpy as jnp
from jax import lax
from jax.experimental import pallas as pl
from jax.experimental.pallas import tpu as pltpu
from jax.experimental.pallas import tpu_sc as plsc

_NC = 2
_NS = 16
_LANES = 16

_CHUNK = 1024
_SUB = 128
_NSUB = _CHUNK // _SUB
_ZROWS = 640
_NRANGE = 2

_G = 64
_BN = 1000


def _ceil_to(a, m):
    return ((a + m - 1) // m) * m


def _fill_const(ref, rows, value):
    def body(r, carry):
        ref[r] = jnp.full((_LANES,), value, jnp.float32)
        return carry

    lax.fori_loop(0, rows, body, None)


def _zero_acc(acc, zerov, span, tile):
    def body(z, carry):
        pltpu.sync_copy(zerov, acc.at[pl.ds(tile * span + z * _ZROWS, _ZROWS)])
        return carry

    lax.fori_loop(0, span // _ZROWS, body, None)


def _clamped_scatter_idx(srcv, sidx, n2, base):
    for j in range(_NSUB):
        def body(u, carry):
            sl = srcv[pl.ds(j * _SUB + u * _LANES, _LANES)] - base
            ok = (sl >= 0) & (sl < n2)
            sidx[j, pl.ds(u * _LANES, _LANES)] = jnp.where(ok, sl, n2)
            return carry

        lax.fori_loop(0, _SUB // _LANES, body, None)


def _mesh():
    return plsc.VectorSubcoreMesh(core_axis_name="c", subcore_axis_name="s",
                                  num_cores=_NC, num_subcores=_NS)


@functools.lru_cache(maxsize=None)
def _build_sc_partition(ep, n2, sent):
    nt = _NC * _NS
    cap = ep // nt
    n_chunks = cap // _CHUNK
    buf = 2 * _CHUNK + _LANES

    @functools.partial(
        pl.kernel,
        out_type=[jax.ShapeDtypeStruct((2, ep), jnp.int32),
                  jax.ShapeDtypeStruct((2, ep), jnp.int32),
                  jax.ShapeDtypeStruct((2, nt * _LANES), jnp.int32)],
        mesh=_mesh(),
        scratch_types=[
            pltpu.VMEM((_CHUNK,), jnp.int32),
            pltpu.VMEM((_CHUNK,), jnp.int32),
            pltpu.VMEM((buf,), jnp.int32),
            pltpu.VMEM((buf,), jnp.int32),
            pltpu.VMEM((buf,), jnp.int32),
            pltpu.VMEM((buf,), jnp.int32),
            pltpu.VMEM((_LANES,), jnp.int32),
        ],
        compiler_params=pltpu.CompilerParams(use_tc_tiling_on_sc=False,
                                             needs_layout_passes=False),
    )
    def part_kernel(src_hbm, dst_hbm, psrc, pdst, cnts,
                    srcv, dstv, bls, bld, bhs, bhd, cntv):
        c = lax.axis_index("c")
        s = lax.axis_index("s")
        w = c * _NS + s
        ebase = w * cap
        sentv = jnp.full((_LANES,), sent, jnp.int32)
        zero = jnp.int32(0)

        def chunk(ci, carry):
            olo, ohi, glo, ghi = carry
            pltpu.sync_copy(src_hbm.at[pl.ds(ebase + ci * _CHUNK, _CHUNK)],
                            srcv)
            pltpu.sync_copy(dst_hbm.at[pl.ds(ebase + ci * _CHUNK, _CHUNK)],
                            dstv)

            def grp(u, car):
                o2lo, o2hi = car
                sv = srcv[pl.ds(u * _LANES, _LANES)]
                dv = dstv[pl.ds(u * _LANES, _LANES)]
                m = sv < n2
                nlo = jnp.max(plsc.all_reduce_population_count(m))
                plsc.store_compressed(bls.at[pl.ds(o2lo, _LANES)], sv, mask=m)
                plsc.store_compressed(bld.at[pl.ds(o2lo, _LANES)], dv, mask=m)
                mh = jnp.logical_not(m)
                plsc.store_compressed(bhs.at[pl.ds(o2hi, _LANES)], sv, mask=mh)
                plsc.store_compressed(bhd.at[pl.ds(o2hi, _LANES)], dv, mask=mh)
                return (o2lo + nlo, o2hi + (_LANES - nlo))

            olo, ohi = lax.fori_loop(0, _CHUNK // _LANES, grp, (olo, ohi))

            fl_lo = olo >= _CHUNK

            @pl.when(fl_lo)
            def _flush_lo():
                pltpu.sync_copy(
                    bls.at[pl.ds(0, _CHUNK)],
                    psrc.at[0, pl.ds(ebase + glo * _CHUNK, _CHUNK)])
                pltpu.sync_copy(
                    bld.at[pl.ds(0, _CHUNK)],
                    pdst.at[0, pl.ds(ebase + glo * _CHUNK, _CHUNK)])

                def mv(u, car2):
                    o = u * _LANES
                    bls[pl.ds(o, _LANES)] = bls[pl.ds(_CHUNK + o, _LANES)]
                    bld[pl.ds(o, _LANES)] = bld[pl.ds(_CHUNK + o, _LANES)]
                    return car2

                lax.fori_loop(0, _CHUNK // _LANES, mv, None)

            olo = jnp.where(fl_lo, olo - _CHUNK, olo)
            glo = glo + fl_lo.astype(jnp.int32)

            fl_hi = ohi >= _CHUNK

            @pl.when(fl_hi)
            def _flush_hi():
                pltpu.sync_copy(
                    bhs.at[pl.ds(0, _CHUNK)],
                    psrc.at[1, pl.ds(ebase + ghi * _CHUNK, _CHUNK)])
                pltpu.sync_copy(
                    bhd.at[pl.ds(0, _CHUNK)],
                    pdst.at[1, pl.ds(ebase + ghi * _CHUNK, _CHUNK)])

                def mv(u, car2):
                    o = u * _LANES
                    bhs[pl.ds(o, _LANES)] = bhs[pl.ds(_CHUNK + o, _LANES)]
                    bhd[pl.ds(o, _LANES)] = bhd[pl.ds(_CHUNK + o, _LANES)]
                    return car2

                lax.fori_loop(0, _CHUNK // _LANES, mv, None)

            ohi = jnp.where(fl_hi, ohi - _CHUNK, ohi)
            ghi = ghi + fl_hi.astype(jnp.int32)
            return (olo, ohi, glo, ghi)

        olo, ohi, glo, ghi = lax.fori_loop(
            0, n_chunks, chunk, (zero, zero, zero, zero))

        def tail_lo(u, car):
            o = olo + u * _LANES
            bls[pl.ds(o, _LANES)] = sentv
            bld[pl.ds(o, _LANES)] = sentv
            return car

        lax.fori_loop(0, _CHUNK // _LANES, tail_lo, None)

        @pl.when(olo > 0)
        def _final_lo():
            pltpu.sync_copy(
                bls.at[pl.ds(0, _CHUNK)],
                psrc.at[0, pl.ds(ebase + glo * _CHUNK, _CHUNK)])
            pltpu.sync_copy(
                bld.at[pl.ds(0, _CHUNK)],
                pdst.at[0, pl.ds(ebase + glo * _CHUNK, _CHUNK)])

        total_lo = glo + (olo > 0).astype(jnp.int32)
        cntv[...] = jnp.broadcast_to(total_lo, (_LANES,))
        pltpu.sync_copy(cntv, cnts.at[0, pl.ds(w * _LANES, _LANES)])

        def tail_hi(u, car):
            o = ohi + u * _LANES
            bhs[pl.ds(o, _LANES)] = sentv
            bhd[pl.ds(o, _LANES)] = sentv
            return car

        lax.fori_loop(0, _CHUNK // _LANES, tail_hi, None)

        @pl.when(ohi > 0)
        def _final_hi():
            pltpu.sync_copy(
                bhs.at[pl.ds(0, _CHUNK)],
                psrc.at[1, pl.ds(ebase + ghi * _CHUNK, _CHUNK)])
            pltpu.sync_copy(
                bhd.at[pl.ds(0, _CHUNK)],
                pdst.at[1, pl.ds(ebase + ghi * _CHUNK, _CHUNK)])

        total_hi = ghi + (ohi > 0).astype(jnp.int32)
        cntv[...] = jnp.broadcast_to(total_hi, (_LANES,))
        pltpu.sync_copy(cntv, cnts.at[1, pl.ds(w * _LANES, _LANES)])

    return part_kernel


@functools.lru_cache(maxsize=None)
def _build_sc_degree(ep, n2, acc_rows):
    nt = _NC * _NS
    cap = ep // nt
    span = acc_rows // _NS

    @functools.partial(
        pl.kernel,
        out_type=jax.ShapeDtypeStruct((_NRANGE, acc_rows, _LANES),
                                      jnp.float32),
        mesh=_mesh(),
        scratch_types=[
            pltpu.VMEM((_CHUNK,), jnp.int32),
            pltpu.VMEM((_NSUB, _SUB), jnp.int32),
            pltpu.VMEM((_LANES,), jnp.int32),
            pltpu.VMEM((_SUB, _LANES), jnp.float32),
            pltpu.VMEM((_ZROWS, _LANES), jnp.float32),
            pltpu.VMEM_SHARED((acc_rows, _LANES), jnp.float32),
        ],
        compiler_params=pltpu.CompilerParams(use_tc_tiling_on_sc=False,
                                             needs_layout_passes=False),
    )
    def deg_kernel(psrc_hbm, cnts_hbm, out_hbm,
                   srcv, sidx, cntv, onesv, zerov, acc):
        c = lax.axis_index("c")
        s = lax.axis_index("s")
        _fill_const(onesv, _SUB, 1.0)
        _fill_const(zerov, _ZROWS, 0.0)
        _zero_acc(acc, zerov, span, s)
        plsc.subcore_barrier()

        for j in range(2):
            w = 2 * s + j
            pltpu.sync_copy(cnts_hbm.at[c, pl.ds(w * _LANES, _LANES)], cntv)
            trips = jnp.max(cntv[...])
            rbase = w * cap

            def chunk(ci, carry):
                pltpu.sync_copy(
                    psrc_hbm.at[c, pl.ds(rbase + ci * _CHUNK, _CHUNK)], srcv)
                _clamped_scatter_idx(srcv, sidx, n2, c * n2)
                for jj in range(_NSUB):
                    pltpu.sync_copy(onesv, acc.at[sidx.at[jj]], add=True)
                return carry

            lax.fori_loop(0, trips, chunk, None)

        plsc.subcore_barrier()
        pltpu.sync_copy(
            acc.at[pl.ds(s * span, span)],
            out_hbm.at[c, pl.ds(s * span, span)],
        )

    return deg_kernel


@functools.lru_cache(maxsize=None)
def _build_sc_segsum(ep, n2, k_groups, acc_rows):
    nt = _NC * _NS
    cap = ep // nt
    span = acc_rows // _NS
    gpc = k_groups // _NC

    @functools.partial(
        pl.kernel,
        out_type=jax.ShapeDtypeStruct((k_groups, _NRANGE, acc_rows, _LANES),
                                      jnp.float32),
        mesh=_mesh(),
        scratch_types=[
            pltpu.VMEM((_CHUNK,), jnp.int32),
            pltpu.VMEM((_CHUNK,), jnp.int32),
            pltpu.VMEM((_CHUNK,), jnp.int32),
            pltpu.VMEM((_NSUB, _SUB), jnp.int32),
            pltpu.VMEM((_LANES,), jnp.int32),
            pltpu.VMEM((_CHUNK, _LANES), jnp.float32),
            pltpu.VMEM((_ZROWS, _LANES), jnp.float32),
            pltpu.VMEM_SHARED((acc_rows, _LANES), jnp.float32),
            pltpu.SemaphoreType.DMA,
        ],
        compiler_params=pltpu.CompilerParams(use_tc_tiling_on_sc=False,
                                             needs_layout_passes=False),
    )
    def seg_kernel(p_hbm, pdst_hbm, psrc_hbm, cnts_hbm, out_hbm,
                   dstv, srcv, gidxv, sidx, cntv, rowsv, zerov, acc, sem):
        c = lax.axis_index("c")
        s = lax.axis_index("s")
        _fill_const(zerov, _ZROWS, 0.0)

        for gi in range(gpc):
            g = c * gpc + gi
            for r in range(_NRANGE):
                _zero_acc(acc, zerov, span, s)
                plsc.subcore_barrier()

                for j in range(2):
                    w = 2 * s + j
                    pltpu.sync_copy(
                        cnts_hbm.at[r, pl.ds(w * _LANES, _LANES)], cntv)
                    trips = jnp.max(cntv[...])
                    rbase = w * cap

                    def chunk(ci, carry):
                        base = rbase + ci * _CHUNK
                        pltpu.sync_copy(
                            pdst_hbm.at[r, pl.ds(base, _CHUNK)], dstv)

                        def mk_idx(u, carry2):
                            o = u * _LANES
                            gidxv[pl.ds(o, _LANES)] = (
                                dstv[pl.ds(o, _LANES)] * k_groups + g)
                            return carry2

                        lax.fori_loop(0, _CHUNK // _LANES, mk_idx, None)
                        gth = pltpu.async_copy(p_hbm.at[gidxv], rowsv, sem)
                        pltpu.sync_copy(
                            psrc_hbm.at[r, pl.ds(base, _CHUNK)], srcv)
                        _clamped_scatter_idx(srcv, sidx, n2, r * n2)
                        gth.wait()
                        for jj in range(_NSUB):
                            pltpu.sync_copy(rowsv.at[pl.ds(jj * _SUB, _SUB)],
                                            acc.at[sidx.at[jj]], add=True)
                        return carry

                    lax.fori_loop(0, trips, chunk, None)

                plsc.subcore_barrier()
                pltpu.sync_copy(
                    acc.at[pl.ds(s * span, span)],
                    out_hbm.at[g, r, pl.ds(s * span, span)],
                )
                plsc.subcore_barrier()

    return seg_kernel


def _k1_body(degp_ref, x_ref, dinv_ref, p1_ref):
    deg = degp_ref[0, :, 0:1]
    dinv = lax.rsqrt(jnp.maximum(deg, 1.0))
    dinv_ref[...] = dinv
    bn, f = x_ref.shape
    fp = p1_ref.shape[1]
    p1_ref[...] = jnp.concatenate(
        [x_ref[...] * dinv, jnp.zeros((bn, fp - f), jnp.float32)], axis=1)


def _k2_body(agg1_ref, dinv_ref, x_ref, w1a_ref, w1b_ref, b1_ref, w2a_ref,
             h1_ref, p2_ref):
    dv = dinv_ref[...]
    a = jnp.concatenate([agg1_ref[kk, 0] for kk in range(2)], axis=1) * dv
    h1 = jnp.dot(a, w1a_ref[...], preferred_element_type=jnp.float32)
    h1 += jnp.dot(x_ref[...], w1b_ref[...], preferred_element_type=jnp.float32)
    h1 = jnp.maximum(h1 + b1_ref[...], 0.0)
    h1_ref[...] = h1
    p2_ref[...] = jnp.dot(h1, w2a_ref[...],
                          preferred_element_type=jnp.float32) * dv


def _k3_body(agg2_ref, dinv_ref, h1_ref, w2b_ref, b2_ref, w3a_ref,
             h2_ref, p3_ref):
    dv = dinv_ref[...]
    h2 = jnp.concatenate([agg2_ref[kk, 0] for kk in range(4)], axis=1) * dv
    h2 += jnp.dot(h1_ref[...], w2b_ref[...], preferred_element_type=jnp.float32)
    h2 = jnp.maximum(h2 + b2_ref[...], 0.0)
    h2_ref[...] = h2
    p3_ref[...] = jnp.dot(h2, w3a_ref[...],
                          preferred_element_type=jnp.float32) * dv


def _k4_body(agg3_ref, dinv_ref, h2_ref, w3b_ref, b3_ref, i_ref,
             s_ref, cnt_ref):
    nb = pl.program_id(0)

    @pl.when(nb == 0)
    def _init():
        s_ref[...] = jnp.zeros_like(s_ref)
        cnt_ref[...] = jnp.zeros_like(cnt_ref)

    h3 = (jnp.concatenate([agg3_ref[kk, 0] for kk in range(2)], axis=1)
          * dinv_ref[...])
    h3 += jnp.dot(h2_ref[...], w3b_ref[...], preferred_element_type=jnp.float32)
    h3 = jnp.maximum(h3 + b3_ref[...], 0.0)
    ib = i_ref[0]
    oh = (lax.broadcasted_iota(jnp.int32, (_G, _BN), 0) == ib).astype(jnp.float32)
    s_ref[...] += jnp.dot(oh, h3, preferred_element_type=jnp.float32)
    cnt_ref[...] += jnp.dot(oh, jnp.ones((_BN, 1), jnp.float32),
                            preferred_element_type=jnp.float32)


def _k5_body(s_ref, cnt_ref, wd_ref, bd_ref, out_ref):
    pooled = s_ref[...] / jnp.maximum(cnt_ref[...], 1.0)
    logits = jnp.dot(pooled, wd_ref[...],
                     preferred_element_type=jnp.float32) + bd_ref[...]
    m = jnp.max(logits, axis=1, keepdims=True)
    e = jnp.exp(logits - m)
    out_ref[...] = e / jnp.sum(e, axis=1, keepdims=True)


def _full(shape):
    return pl.BlockSpec(shape, lambda nb: tuple(0 for _ in shape))


def _rows(bn, cols):
    return pl.BlockSpec((bn, cols), lambda nb: (nb, 0))


def _planes(k, bn, nbh):
    return pl.BlockSpec((k, 1, bn, _LANES),
                        lambda nb: (0, nb // nbh, nb % nbh, 0))


def kernel(x, edge_index, i, W1a, W1b, b1, W2a, W2b, b2, W3a, W3b, b3, Wd, bd):
    n, f = x.shape
    e = edge_index.shape[1]
    f1 = W1a.shape[1]
    f2 = W2a.shape[1]
    f3 = W3a.shape[1]
    fp = _ceil_to(f, 2 * _LANES)

    ep = _ceil_to(e, _NC * _NS * _CHUNK * 2)
    n2 = n // _NRANGE
    acc_rows = _ceil_to(n2 + 1, _NS * _ZROWS)
    nb = n // _BN
    nbh = n2 // _BN
    assert n % (_NRANGE * _BN) == 0

    src = edge_index[0]
    dst = edge_index[1]
    src_p = jnp.concatenate([src, jnp.full((ep - e,), n, jnp.int32)])
    dst_p = jnp.concatenate([dst, jnp.full((ep - e,), n, jnp.int32)])

    w1a_p = jnp.pad(W1a, ((0, fp - f), (0, 0)))

    psrc, pdst, cnts = _build_sc_partition(ep, n2, n)(src_p, dst_p)

    degp = _build_sc_degree(ep, n2, acc_rows)(psrc, cnts)

    dinv, p1 = pl.pallas_call(
        _k1_body,
        grid=(nb,),
        in_specs=[pl.BlockSpec((1, _BN, _LANES),
                               lambda nb: (nb // nbh, nb % nbh, 0)),
                  _rows(_BN, f)],
        out_specs=[_rows(_BN, 1), _rows(_BN, fp)],
        out_shape=[jax.ShapeDtypeStruct((n, 1), jnp.float32),
                   jax.ShapeDtypeStruct((n + 1, fp), jnp.float32)],
    )(degp, x)

    k1g = fp // _LANES
    agg1 = _build_sc_segsum(ep, n2, k1g, acc_rows)(
        p1.reshape((n + 1) * k1g, _LANES), pdst, psrc, cnts)

    h1, p2 = pl.pallas_call(
        _k2_body,
        grid=(nb,),
        in_specs=[_planes(k1g, _BN, nbh), _rows(_BN, 1), _rows(_BN, f),
                  _full((fp, f1)), _full((f, f1)), _full((1, f1)),
                  _full((f1, f2))],
        out_specs=[_rows(_BN, f1), _rows(_BN, f2)],
        out_shape=[jax.ShapeDtypeStruct((n, f1), jnp.float32),
                   jax.ShapeDtypeStruct((n + 1, f2), jnp.float32)],
    )(agg1, dinv, x, w1a_p, W1b, b1.reshape(1, f1), W2a)

    k2g = f2 // _LANES
    agg2 = _build_sc_segsum(ep, n2, k2g, acc_rows)(
        p2.reshape((n + 1) * k2g, _LANES), pdst, psrc, cnts)

    h2, p3 = pl.pallas_call(
        _k3_body,
        grid=(nb,),
        in_specs=[_planes(k2g, _BN, nbh), _rows(_BN, 1), _rows(_BN, f1),
                  _full((f1, f2)), _full((1, f2)), _full((f2, f3))],
        out_specs=[_rows(_BN, f2), _rows(_BN, f3)],
        out_shape=[jax.ShapeDtypeStruct((n, f2), jnp.float32),
                   jax.ShapeDtypeStruct((n + 1, f3), jnp.float32)],
    )(agg2, dinv, h1, W2b, b2.reshape(1, f2), W3a)

    k3g = f3 // _LANES
    agg3 = _build_sc_segsum(ep, n2, k3g, acc_rows)(
        p3.reshape((n + 1) * k3g, _LANES), pdst, psrc, cnts)

    s_sum, cnt = pl.pallas_call(
        _k4_body,
        grid=(nb,),
        in_specs=[_planes(k3g, _BN, nbh), _rows(_BN, 1), _rows(_BN, f2),
                  _full((f2, f3)), _full((1, f3)),
                  pl.BlockSpec((1, 1, _BN), lambda nb: (nb, 0, 0))],
        out_specs=[_full((_G, f3)), _full((_G, 1))],
        out_shape=[jax.ShapeDtypeStruct((_G, f3), jnp.float32),
                   jax.ShapeDtypeStruct((_G, 1), jnp.float32)],
    )(agg3, dinv, h2, W3b, b3.reshape(1, f3), i.reshape(nb, 1, _BN))

    out = pl.pallas_call(
        _k5_body,
        grid=(1,),
        in_specs=[_full((_G, f3)), _full((_G, 1)),
                  _full((f3, Wd.shape[1])), _full((1, Wd.shape[1]))],
        out_specs=_full((_G, Wd.shape[1])),
        out_shape=jax.ShapeDtypeStruct((_G, Wd.shape[1]), jnp.float32),
    )(s_sum, cnt, Wd, bd.reshape(1, Wd.shape[1]))

    return out
